# baseline (speedup 1.0000x reference)
import functools

import jax
import jax.numpy as jnp
from jax import lax
from jax.experimental import pallas as pl
from jax.experimental.pallas import tpu as pltpu
from jax.experimental.pallas import tpu_sc as plsc

_B, _S, _D = 4, 4096, 1024
_NC, _NS, _L = 2, 16, 16
_NW = _NC * _NS
_ROWS_PER_W = _S // _NW
_CH = 32
_NCHUNK = _ROWS_PER_W // _CH


def _sc_body(x_hbm, pos_hbm, out_hbm, pos_v, x_v):
    wid = lax.axis_index("s") * _NC + lax.axis_index("c")
    base = wid * _ROWS_PER_W

    def do_chunk(c, _):
        row0 = base + c * _CH
        pltpu.sync_copy(pos_hbm.at[pl.ds(row0, _CH)], pos_v)
        for b in range(_B):
            pltpu.sync_copy(x_hbm.at[b, pl.ds(row0, _CH)], x_v)

            def do_row(r, _):
                for g in range(_D // _L):
                    sl = pl.ds(g * _L, _L)
                    x_v[r, sl] = x_v[r, sl] + pos_v[r, sl]
                return 0

            lax.fori_loop(0, _CH, do_row, 0)
            pltpu.sync_copy(x_v, out_hbm.at[b, pl.ds(row0, _CH)])
        return 0

    lax.fori_loop(0, _NCHUNK, do_chunk, 0)


def kernel(x, pos_table):
    mesh = plsc.VectorSubcoreMesh(core_axis_name="c", subcore_axis_name="s")
    k = functools.partial(
        pl.kernel,
        mesh=mesh,
        out_type=jax.ShapeDtypeStruct((_B, _S, _D), jnp.float32),
        scratch_types=[
            pltpu.VMEM((_CH, _D), jnp.float32),
            pltpu.VMEM((_CH, _D), jnp.float32),
        ],
    )(_sc_body)
    return k(x, pos_table[:_S])
